# baseline (speedup 1.0000x reference)
import jax
import jax.numpy as jnp
from jax.experimental import pallas as pl
from jax.experimental.pallas import tpu as pltpu


def _shuffle_kernel(pos_ref, emb_ref, hs_ref, mask_ref,
                    out_hs_ref, out_mask_ref,
                    sem_big, sem_ctx):
    batch = out_hs_ref.shape[0]
    n_ctx = emb_ref.shape[1]
    seq = hs_ref.shape[1]

    n_chunks = 8
    chunk = seq // n_chunks
    big_copies = []
    for b in range(batch):
        for c in range(n_chunks):
            cp = pltpu.make_async_copy(
                hs_ref.at[b, pl.ds(c * chunk, chunk), :],
                out_hs_ref.at[b, pl.ds(n_ctx + c * chunk, chunk), :],
                sem_big)
            cp.start()
            big_copies.append(cp)

    p = pos_ref[0]
    ctx_copies = []
    for b in range(batch):
        c = pltpu.make_async_copy(
            emb_ref.at[p], out_hs_ref.at[b, pl.ds(0, n_ctx), :], sem_ctx)
        c.start()
        ctx_copies.append(c)

    out_mask_ref[:, :n_ctx] = jnp.ones_like(out_mask_ref[:, :n_ctx])
    out_mask_ref[:, n_ctx:] = mask_ref[:]

    for cp in big_copies:
        cp.wait()
    for c in ctx_copies:
        c.wait()


def kernel(hidden_states, attention_mask, position, embeddings):
    B, S, H = hidden_states.shape
    T = embeddings.shape[1]
    pos = jnp.asarray(position, dtype=jnp.int32).reshape((1,))

    out_hs, out_mask = pl.pallas_call(
        _shuffle_kernel,
        in_specs=[
            pl.BlockSpec(memory_space=pltpu.SMEM),
            pl.BlockSpec(memory_space=pl.ANY),
            pl.BlockSpec(memory_space=pl.ANY),
            pl.BlockSpec(memory_space=pltpu.VMEM),
        ],
        out_specs=[
            pl.BlockSpec(memory_space=pl.ANY),
            pl.BlockSpec(memory_space=pltpu.VMEM),
        ],
        out_shape=[
            jax.ShapeDtypeStruct((B, T + S, H), hidden_states.dtype),
            jax.ShapeDtypeStruct((B, T + S), attention_mask.dtype),
        ],
        scratch_shapes=[pltpu.SemaphoreType.DMA, pltpu.SemaphoreType.DMA],
    )(pos, embeddings, hidden_states, attention_mask)
    return out_hs, out_mask


# speedup vs baseline: 46.3616x; 46.3616x over previous
import jax
import jax.numpy as jnp
from jax.experimental import pallas as pl
from jax.experimental.pallas import tpu as pltpu

_SEQ_CHUNK = 2048


def _shuffle_kernel(pos_ref, emb_ref, hs_ref, mask_ref,
                    out_hs_ref, out_mask_ref,
                    sem_big, sem_ctx):
    b = pl.program_id(0)
    k = pl.program_id(1)
    n_ctx = emb_ref.shape[1]
    chunk = hs_ref.shape[1]

    cp = pltpu.make_async_copy(
        hs_ref,
        out_hs_ref.at[pl.ds(b, 1), pl.ds(n_ctx + k * chunk, chunk), :],
        sem_big)
    cp.start()

    @pl.when(k == 0)
    def _():
        p = pos_ref[0]
        ctx = pltpu.make_async_copy(
            emb_ref.at[pl.ds(p, 1)],
            out_hs_ref.at[pl.ds(b, 1), pl.ds(0, n_ctx), :], sem_ctx)
        ctx.start()

        @pl.when(b == 0)
        def _():
            out_mask_ref[:, :n_ctx] = jnp.ones_like(out_mask_ref[:, :n_ctx])
            out_mask_ref[:, n_ctx:] = mask_ref[:, :]

        ctx.wait()

    cp.wait()


def kernel(hidden_states, attention_mask, position, embeddings):
    B, S, H = hidden_states.shape
    D, T, _ = embeddings.shape
    pos = jnp.asarray(position, dtype=jnp.int32).reshape((1,))

    out_hs, out_mask = pl.pallas_call(
        _shuffle_kernel,
        grid=(B, S // _SEQ_CHUNK),
        in_specs=[
            pl.BlockSpec(memory_space=pltpu.SMEM),
            pl.BlockSpec((D, T, H), lambda b, k: (0, 0, 0)),
            pl.BlockSpec((1, _SEQ_CHUNK, H), lambda b, k: (b, k, 0)),
            pl.BlockSpec((B, S), lambda b, k: (0, 0)),
        ],
        out_specs=[
            pl.BlockSpec(memory_space=pl.ANY),
            pl.BlockSpec((B, T + S), lambda b, k: (0, 0)),
        ],
        out_shape=[
            jax.ShapeDtypeStruct((B, T + S, H), hidden_states.dtype),
            jax.ShapeDtypeStruct((B, T + S), attention_mask.dtype),
        ],
        scratch_shapes=[pltpu.SemaphoreType.DMA, pltpu.SemaphoreType.DMA],
    )(pos, embeddings, hidden_states, attention_mask)
    return out_hs, out_mask


# speedup vs baseline: 47.4114x; 1.0226x over previous
import jax
import jax.numpy as jnp
from jax.experimental import pallas as pl
from jax.experimental.pallas import tpu as pltpu

_SEQ_CHUNK = 1024
_NBUF = 4
_LOOKAHEAD = 2


def _shuffle_kernel(pos_ref, emb_ref, hs_ref, mask_ref,
                    out_hs_ref, out_mask_ref,
                    buf, sem_in, sem_out, sem_ctx):
    batch, seq, _ = hs_ref.shape
    n_ctx = emb_ref.shape[1]
    chunk = _SEQ_CHUNK
    kpb = seq // chunk
    n = batch * kpb

    def fetch(i):
        b, k = divmod(i, kpb)
        m = i % _NBUF
        return pltpu.make_async_copy(
            hs_ref.at[pl.ds(b, 1), pl.ds(k * chunk, chunk), :],
            buf.at[pl.ds(m, 1)], sem_in.at[m])

    def write(i):
        b, k = divmod(i, kpb)
        m = i % _NBUF
        return pltpu.make_async_copy(
            buf.at[pl.ds(m, 1)],
            out_hs_ref.at[pl.ds(b, 1), pl.ds(n_ctx + k * chunk, chunk), :],
            sem_out.at[m])

    def ctx_copy(b2):
        p = pos_ref[0]
        return pltpu.make_async_copy(
            emb_ref.at[pl.ds(p, 1)],
            out_hs_ref.at[pl.ds(b2, 1), pl.ds(0, n_ctx), :], sem_ctx)

    for b2 in range(batch):
        ctx_copy(b2).start()

    for i in range(_LOOKAHEAD):
        fetch(i).start()

    for i in range(n):
        j = i + _LOOKAHEAD
        if j < n:
            if j - _NBUF >= 0:
                write(j - _NBUF).wait()
            fetch(j).start()
        fetch(i).wait()
        write(i).start()

    out_mask_ref[:, :n_ctx] = jnp.ones_like(out_mask_ref[:, :n_ctx])
    out_mask_ref[:, n_ctx:] = mask_ref[:, :]

    for i in range(max(0, n - _NBUF), n):
        write(i).wait()
    for b2 in range(batch):
        ctx_copy(b2).wait()


def kernel(hidden_states, attention_mask, position, embeddings):
    B, S, H = hidden_states.shape
    D, T, _ = embeddings.shape
    pos = jnp.asarray(position, dtype=jnp.int32).reshape((1,))

    out_hs, out_mask = pl.pallas_call(
        _shuffle_kernel,
        in_specs=[
            pl.BlockSpec(memory_space=pltpu.SMEM),
            pl.BlockSpec(memory_space=pl.ANY),
            pl.BlockSpec(memory_space=pl.ANY),
            pl.BlockSpec(memory_space=pltpu.VMEM),
        ],
        out_specs=[
            pl.BlockSpec(memory_space=pl.ANY),
            pl.BlockSpec(memory_space=pltpu.VMEM),
        ],
        out_shape=[
            jax.ShapeDtypeStruct((B, T + S, H), hidden_states.dtype),
            jax.ShapeDtypeStruct((B, T + S), attention_mask.dtype),
        ],
        scratch_shapes=[
            pltpu.VMEM((_NBUF, _SEQ_CHUNK, H), hidden_states.dtype),
            pltpu.SemaphoreType.DMA((_NBUF,)),
            pltpu.SemaphoreType.DMA((_NBUF,)),
            pltpu.SemaphoreType.DMA,
        ],
    )(pos, embeddings, hidden_states, attention_mask)
    return out_hs, out_mask


# speedup vs baseline: 47.4440x; 1.0007x over previous
import jax
import jax.numpy as jnp
from jax.experimental import pallas as pl
from jax.experimental.pallas import tpu as pltpu

_SEQ_CHUNK = 512
_NBUF = 6
_LOOKAHEAD = 3


def _shuffle_kernel(pos_ref, emb_ref, hs_ref, mask_ref,
                    out_hs_ref, out_mask_ref,
                    buf, sem_in, sem_out, sem_ctx):
    batch, seq, _ = hs_ref.shape
    n_ctx = emb_ref.shape[1]
    chunk = _SEQ_CHUNK
    kpb = seq // chunk
    n = batch * kpb

    def fetch(i):
        b, k = divmod(i, kpb)
        m = i % _NBUF
        return pltpu.make_async_copy(
            hs_ref.at[pl.ds(b, 1), pl.ds(k * chunk, chunk), :],
            buf.at[pl.ds(m, 1)], sem_in.at[m])

    def write(i):
        b, k = divmod(i, kpb)
        m = i % _NBUF
        return pltpu.make_async_copy(
            buf.at[pl.ds(m, 1)],
            out_hs_ref.at[pl.ds(b, 1), pl.ds(n_ctx + k * chunk, chunk), :],
            sem_out.at[m])

    def ctx_copy(b2):
        p = pos_ref[0]
        return pltpu.make_async_copy(
            emb_ref.at[pl.ds(p, 1)],
            out_hs_ref.at[pl.ds(b2, 1), pl.ds(0, n_ctx), :], sem_ctx)

    for b2 in range(batch):
        ctx_copy(b2).start()

    for i in range(_LOOKAHEAD):
        fetch(i).start()

    for i in range(n):
        j = i + _LOOKAHEAD
        if j < n:
            if j - _NBUF >= 0:
                write(j - _NBUF).wait()
            fetch(j).start()
        fetch(i).wait()
        write(i).start()

    out_mask_ref[:, :n_ctx] = jnp.ones_like(out_mask_ref[:, :n_ctx])
    out_mask_ref[:, n_ctx:] = mask_ref[:, :]

    for i in range(max(0, n - _NBUF), n):
        write(i).wait()
    for b2 in range(batch):
        ctx_copy(b2).wait()


def kernel(hidden_states, attention_mask, position, embeddings):
    B, S, H = hidden_states.shape
    D, T, _ = embeddings.shape
    pos = jnp.asarray(position, dtype=jnp.int32).reshape((1,))

    out_hs, out_mask = pl.pallas_call(
        _shuffle_kernel,
        in_specs=[
            pl.BlockSpec(memory_space=pltpu.SMEM),
            pl.BlockSpec(memory_space=pl.ANY),
            pl.BlockSpec(memory_space=pl.ANY),
            pl.BlockSpec(memory_space=pltpu.VMEM),
        ],
        out_specs=[
            pl.BlockSpec(memory_space=pl.ANY),
            pl.BlockSpec(memory_space=pltpu.VMEM),
        ],
        out_shape=[
            jax.ShapeDtypeStruct((B, T + S, H), hidden_states.dtype),
            jax.ShapeDtypeStruct((B, T + S), attention_mask.dtype),
        ],
        scratch_shapes=[
            pltpu.VMEM((_NBUF, _SEQ_CHUNK, H), hidden_states.dtype),
            pltpu.SemaphoreType.DMA((_NBUF,)),
            pltpu.SemaphoreType.DMA((_NBUF,)),
            pltpu.SemaphoreType.DMA,
        ],
    )(pos, embeddings, hidden_states, attention_mask)
    return out_hs, out_mask


# speedup vs baseline: 47.7624x; 1.0067x over previous
import jax
import jax.numpy as jnp
from jax.experimental import pallas as pl
from jax.experimental.pallas import tpu as pltpu

_MID_CHUNK = 1024
_RAMP = [128, 128, 256, 512]
_NBUF = 6
_LOOKAHEAD = 4


def _chunk_schedule(batch, seq):
    chunks = []
    for b in range(batch):
        sizes = []
        if b == 0:
            sizes += _RAMP
        tail = sum(_RAMP) if b == batch - 1 else 0
        n_mid = (seq - sum(sizes) - tail) // _MID_CHUNK
        sizes += [_MID_CHUNK] * n_mid
        if b == batch - 1:
            sizes += list(reversed(_RAMP))
        row = 0
        for s in sizes:
            chunks.append((b, row, s))
            row += s
    return chunks


def _shuffle_kernel(pos_ref, emb_ref, hs_ref, mask_ref,
                    out_hs_ref, out_mask_ref,
                    buf, sem_in, sem_out, sem_ctx):
    batch, seq, _ = hs_ref.shape
    n_ctx = emb_ref.shape[1]
    sched = _chunk_schedule(batch, seq)
    n = len(sched)

    def fetch(i):
        b, row, sz = sched[i]
        m = i % _NBUF
        return pltpu.make_async_copy(
            hs_ref.at[pl.ds(b, 1), pl.ds(row, sz), :],
            buf.at[pl.ds(m, 1), pl.ds(0, sz), :], sem_in.at[m])

    def write(i):
        b, row, sz = sched[i]
        m = i % _NBUF
        return pltpu.make_async_copy(
            buf.at[pl.ds(m, 1), pl.ds(0, sz), :],
            out_hs_ref.at[pl.ds(b, 1), pl.ds(n_ctx + row, sz), :],
            sem_out.at[m])

    def ctx_copy(b2):
        p = pos_ref[0]
        return pltpu.make_async_copy(
            emb_ref.at[pl.ds(p, 1)],
            out_hs_ref.at[pl.ds(b2, 1), pl.ds(0, n_ctx), :], sem_ctx)

    for b2 in range(batch):
        ctx_copy(b2).start()

    for i in range(_LOOKAHEAD):
        fetch(i).start()

    for i in range(n):
        j = i + _LOOKAHEAD
        if j < n:
            if j - _NBUF >= 0:
                write(j - _NBUF).wait()
            fetch(j).start()
        fetch(i).wait()
        write(i).start()

    out_mask_ref[:, :n_ctx] = jnp.ones_like(out_mask_ref[:, :n_ctx])
    out_mask_ref[:, n_ctx:] = mask_ref[:, :]

    for i in range(max(0, n - _NBUF), n):
        write(i).wait()
    for b2 in range(batch):
        ctx_copy(b2).wait()


def kernel(hidden_states, attention_mask, position, embeddings):
    B, S, H = hidden_states.shape
    D, T, _ = embeddings.shape
    pos = jnp.asarray(position, dtype=jnp.int32).reshape((1,))

    out_hs, out_mask = pl.pallas_call(
        _shuffle_kernel,
        in_specs=[
            pl.BlockSpec(memory_space=pltpu.SMEM),
            pl.BlockSpec(memory_space=pl.ANY),
            pl.BlockSpec(memory_space=pl.ANY),
            pl.BlockSpec(memory_space=pltpu.VMEM),
        ],
        out_specs=[
            pl.BlockSpec(memory_space=pl.ANY),
            pl.BlockSpec(memory_space=pltpu.VMEM),
        ],
        out_shape=[
            jax.ShapeDtypeStruct((B, T + S, H), hidden_states.dtype),
            jax.ShapeDtypeStruct((B, T + S), attention_mask.dtype),
        ],
        scratch_shapes=[
            pltpu.VMEM((_NBUF, _MID_CHUNK, H), hidden_states.dtype),
            pltpu.SemaphoreType.DMA((_NBUF,)),
            pltpu.SemaphoreType.DMA((_NBUF,)),
            pltpu.SemaphoreType.DMA,
        ],
    )(pos, embeddings, hidden_states, attention_mask)
    return out_hs, out_mask
